# baseline (speedup 1.0000x reference)
import numpy as np

import jax
import jax.numpy as jnp
from jax.experimental import pallas as pl
from jax.experimental.pallas import tpu as pltpu

_H1, _C1, _OC1 = 32, 3, 32
_H2, _C2, _OC2 = 15, 32, 64
_H3, _C3, _OC3 = 6, 64, 64
_FC1, _FC2 = 128, 10


def _banded(w_taps, h, c, oc, cmajor=False):
    ow = h - 2
    owp = ow // 2
    perm = [2 * k for k in range(owp)] + [2 * k + 1 for k in range(owp)]
    if ow % 2:
        perm.append(ow - 1)
    e = np.zeros((9, 3, h, ow), np.float32)
    for d in range(3):
        for dj in range(3):
            for jp, j in enumerate(perm):
                e[d * 3 + dj, d, j + dj, jp] = 1.0
    spec = 'tco,tdhj->dchjo' if cmajor else 'tco,tdhj->dhcjo'
    wb = jnp.einsum(spec, w_taps, jnp.asarray(e))
    return wb.reshape(3, h * c, ow * oc)


def _cnn_kernel(x_ref, w1_ref, b1_ref, w2_ref, b2_ref, w3_ref, b3_ref,
                fw1_ref, fb1_ref, fw2_ref, fb2_ref, o_ref, s2_ref, s3_ref):
    b = x_ref.shape[1]

    def conv_pool(src_ref, wb_ref, bias, rows, half, fused_k=False):
        if fused_k:
            sl = src_ref[0:rows, :, :].reshape(rows * b, -1)
            acc = jnp.dot(sl, wb_ref[...],
                          preferred_element_type=jnp.float32)
        else:
            acc = None
            for di in range(3):
                sl = src_ref[di:di + rows, :, :].reshape(rows * b, -1)
                d = jnp.dot(sl, wb_ref[di],
                            preferred_element_type=jnp.float32)
                acc = d if acc is None else acc + d
        n = acc.shape[-1]
        acc = acc.reshape(rows // 2, 2 * b, n)
        rm = jnp.maximum(acc[:, :b, :], acc[:, b:, :])
        cm = jnp.maximum(rm[..., :half], rm[..., half:2 * half])
        return jnp.maximum(cm + bias, 0.0).astype(jnp.bfloat16)

    s2_ref[...] = conv_pool(x_ref, w1_ref, b1_ref[...], 30, 480, fused_k=True)
    s3_ref[...] = conv_pool(s2_ref, w2_ref, b2_ref[...], 12, 384)
    p3 = conv_pool(s3_ref, w3_ref, b3_ref[...], 4, 128)

    flat = jnp.concatenate([p3[0], p3[1]], axis=1)
    h = jnp.dot(flat, fw1_ref[...], preferred_element_type=jnp.float32)
    h = jnp.maximum(h + fb1_ref[...], 0.0).astype(jnp.bfloat16)
    logits = jnp.dot(h, fw2_ref[...].astype(jnp.bfloat16), preferred_element_type=jnp.float32)
    o_ref[...] = logits + fb2_ref[...]


def kernel(w1, b1, w2, b2, w3, b3, fw1, fb1, fw2, fb2, x):
    n = x.shape[0]
    bsz = next(b for b in (128, 64, 32, 16, 8, 4, 2, 1) if n % b == 0)

    xp = jnp.transpose(x.astype(jnp.bfloat16), (2, 0, 3, 1)).reshape(_H1, n, _H1 * _C1)
    xcat = jnp.concatenate([xp[0:30], xp[1:31], xp[2:32]], axis=2)

    w1b = _banded(w1, _H1, _C1, _OC1).astype(jnp.bfloat16).reshape(3 * _H1 * _C1, 30 * _OC1)
    w2b = _banded(w2, _H2, _C2, _OC2).astype(jnp.bfloat16)
    w3b = _banded(w3, _H3, _C3, _OC3).astype(jnp.bfloat16)
    b1t = jnp.tile(b1, (1, 15))
    b2t = jnp.tile(b2, (1, 6))
    b3t = jnp.tile(b3, (1, 2))
    fw1r = fw1.reshape(4 * _OC3, _FC1).astype(jnp.bfloat16)

    full2 = lambda i: (0, 0)
    full3 = lambda i: (0, 0, 0)
    out = pl.pallas_call(
        _cnn_kernel,
        out_shape=jax.ShapeDtypeStruct((n, _FC2), jnp.float32),
        grid_spec=pltpu.PrefetchScalarGridSpec(
            num_scalar_prefetch=0,
            grid=(n // bsz,),
            in_specs=[
                pl.BlockSpec((30, bsz, 3 * _H1 * _C1), lambda i: (0, i, 0)),
                pl.BlockSpec((3 * _H1 * _C1, 30 * _OC1), full2),
                pl.BlockSpec((1, 15 * _OC1), full2),
                pl.BlockSpec((3, _H2 * _C2, 13 * _OC2), full3),
                pl.BlockSpec((1, 6 * _OC2), full2),
                pl.BlockSpec((3, _H3 * _C3, 4 * _OC3), full3),
                pl.BlockSpec((1, 2 * _OC3), full2),
                pl.BlockSpec((4 * _OC3, _FC1), full2),
                pl.BlockSpec((1, _FC1), full2),
                pl.BlockSpec((_FC1, _FC2), full2),
                pl.BlockSpec((1, _FC2), full2),
            ],
            out_specs=pl.BlockSpec((bsz, _FC2), lambda i: (i, 0)),
            scratch_shapes=[
                pltpu.VMEM((15, bsz, 15 * _OC1), jnp.bfloat16),
                pltpu.VMEM((6, bsz, 6 * _OC2), jnp.bfloat16),
            ],
        ),
        compiler_params=pltpu.CompilerParams(
            dimension_semantics=("parallel",)),
    )(xcat, w1b, b1t, w2b, b2t, w3b, b3t, fw1r, fb1, fw2, fb2)
    return out


# speedup vs baseline: 1.2674x; 1.2674x over previous
import numpy as np

import jax
import jax.numpy as jnp
from jax.experimental import pallas as pl
from jax.experimental.pallas import tpu as pltpu

_H1, _C1, _OC1 = 32, 3, 32
_H2, _C2, _OC2 = 15, 32, 64
_H3, _C3, _OC3 = 6, 64, 64
_FC1, _FC2 = 128, 10


def _banded(w_taps, h, c, oc, cmajor=False):
    ow = h - 2
    owp = ow // 2
    perm = [2 * k for k in range(owp)] + [2 * k + 1 for k in range(owp)]
    if ow % 2:
        perm.append(ow - 1)
    e = np.zeros((9, 3, h, ow), np.float32)
    for d in range(3):
        for dj in range(3):
            for jp, j in enumerate(perm):
                e[d * 3 + dj, d, j + dj, jp] = 1.0
    spec = 'tco,tdhj->dchjo' if cmajor else 'tco,tdhj->dhcjo'
    wb = jnp.einsum(spec, w_taps, jnp.asarray(e))
    return wb.reshape(3, h * c, ow * oc)


def _cnn_kernel(x_ref, w1_ref, b1_ref, w2_ref, b2_ref, w3_ref, b3_ref,
                fw1_ref, fb1_ref, fw2_ref, fb2_ref, o_ref, s2_ref, s3_ref):
    b = x_ref.shape[1]

    def conv_pool(src_ref, wb_ref, bias, rows, half):
        acc = None
        for di in range(3):
            sl = src_ref[di:di + rows, :, :].reshape(rows * b, -1)
            d = jnp.dot(sl, wb_ref[di],
                        preferred_element_type=jnp.float32)
            acc = d if acc is None else acc + d
        n = acc.shape[-1]
        acc = acc.reshape(rows // 2, 2 * b, n)
        rm = jnp.maximum(acc[:, :b, :], acc[:, b:, :])
        cm = jnp.maximum(rm[..., :half], rm[..., half:2 * half])
        return jnp.maximum(cm + bias, 0.0).astype(jnp.bfloat16)

    s2_ref[...] = conv_pool(x_ref, w1_ref, b1_ref[...], 30, 480)
    s3_ref[...] = conv_pool(s2_ref, w2_ref, b2_ref[...], 12, 384)
    p3 = conv_pool(s3_ref, w3_ref, b3_ref[...], 4, 128)

    flat = jnp.concatenate([p3[0], p3[1]], axis=1)
    h = jnp.dot(flat, fw1_ref[...], preferred_element_type=jnp.float32)
    h = jnp.maximum(h + fb1_ref[...], 0.0).astype(jnp.bfloat16)
    logits = jnp.dot(h, fw2_ref[...].astype(jnp.bfloat16), preferred_element_type=jnp.float32)
    o_ref[...] = logits + fb2_ref[...]


def kernel(w1, b1, w2, b2, w3, b3, fw1, fb1, fw2, fb2, x):
    n = x.shape[0]
    bsz = next(b for b in (128, 64, 32, 16, 8, 4, 2, 1) if n % b == 0)

    xp = jnp.transpose(x.astype(jnp.bfloat16), (2, 0, 3, 1)).reshape(_H1, n, _H1 * _C1)

    w1b = _banded(w1, _H1, _C1, _OC1).astype(jnp.bfloat16)
    w2b = _banded(w2, _H2, _C2, _OC2).astype(jnp.bfloat16)
    w3b = _banded(w3, _H3, _C3, _OC3).astype(jnp.bfloat16)
    b1t = jnp.tile(b1, (1, 15))
    b2t = jnp.tile(b2, (1, 6))
    b3t = jnp.tile(b3, (1, 2))
    fw1r = fw1.reshape(4 * _OC3, _FC1).astype(jnp.bfloat16)

    full2 = lambda i: (0, 0)
    full3 = lambda i: (0, 0, 0)
    out = pl.pallas_call(
        _cnn_kernel,
        out_shape=jax.ShapeDtypeStruct((n, _FC2), jnp.float32),
        grid_spec=pltpu.PrefetchScalarGridSpec(
            num_scalar_prefetch=0,
            grid=(n // bsz,),
            in_specs=[
                pl.BlockSpec((_H1, bsz, _H1 * _C1), lambda i: (0, i, 0)),
                pl.BlockSpec((3, _H1 * _C1, 30 * _OC1), full3),
                pl.BlockSpec((1, 15 * _OC1), full2),
                pl.BlockSpec((3, _H2 * _C2, 13 * _OC2), full3),
                pl.BlockSpec((1, 6 * _OC2), full2),
                pl.BlockSpec((3, _H3 * _C3, 4 * _OC3), full3),
                pl.BlockSpec((1, 2 * _OC3), full2),
                pl.BlockSpec((4 * _OC3, _FC1), full2),
                pl.BlockSpec((1, _FC1), full2),
                pl.BlockSpec((_FC1, _FC2), full2),
                pl.BlockSpec((1, _FC2), full2),
            ],
            out_specs=pl.BlockSpec((bsz, _FC2), lambda i: (i, 0)),
            scratch_shapes=[
                pltpu.VMEM((15, bsz, 15 * _OC1), jnp.bfloat16),
                pltpu.VMEM((6, bsz, 6 * _OC2), jnp.bfloat16),
            ],
        ),
        compiler_params=pltpu.CompilerParams(
            dimension_semantics=("parallel",)),
    )(xp, w1b, b1t, w2b, b2t, w3b, b3t, fw1r, fb1, fw2, fb2)
    return out


# speedup vs baseline: 1.3724x; 1.0829x over previous
import numpy as np

import jax
import jax.numpy as jnp
from jax.experimental import pallas as pl
from jax.experimental.pallas import tpu as pltpu

_H1, _C1, _OC1 = 32, 3, 32
_H2, _C2, _OC2 = 15, 32, 64
_H3, _C3, _OC3 = 6, 64, 64
_FC1, _FC2 = 128, 10
_W1P, _W2P, _W3P = 128, 512, 384


def _banded(w_taps, h, c, oc):
    ow = h - 2
    owp = ow // 2
    perm = [2 * k for k in range(owp)] + [2 * k + 1 for k in range(owp)]
    if ow % 2:
        perm.append(ow - 1)
    e = np.zeros((9, 3, h, ow), np.float32)
    for d in range(3):
        for dj in range(3):
            for jp, j in enumerate(perm):
                e[d * 3 + dj, d, j + dj, jp] = 1.0
    wb = jnp.einsum('tco,tdhj->dhcjo', w_taps, jnp.asarray(e))
    return wb.reshape(3, h * c, ow * oc)


def _pad_k(wb3, kp):
    k = wb3.shape[1]
    return jnp.pad(wb3, ((0, 0), (0, kp - k), (0, 0))).reshape(
        3 * kp, wb3.shape[2]).astype(jnp.bfloat16)


def _cnn_kernel(x_ref, w1_ref, b1_ref, w2_ref, b2_ref, w3_ref, b3_ref,
                fw1_ref, fb1_ref, fw2_ref, fb2_ref, o_ref,
                sx1_ref, s2_ref, sx2_ref, s3_ref, sx3_ref):
    b = x_ref.shape[1]

    def fill3(dst_ref, src_ref, rows, w):
        for di in range(3):
            dst_ref[:, :, di * w:(di + 1) * w] = src_ref[di:di + rows, :, :]

    def conv_pool(cat_ref, wb_ref, bias, rows, half):
        sl = cat_ref[...].reshape(rows * b, cat_ref.shape[2])
        acc = jnp.dot(sl, wb_ref[...], preferred_element_type=jnp.float32)
        n = acc.shape[-1]
        acc = acc.reshape(rows // 2, 2 * b, n)
        rm = jnp.maximum(acc[:, :b, :], acc[:, b:, :])
        cm = jnp.maximum(rm[..., :half], rm[..., half:2 * half])
        return jnp.maximum(cm + bias, 0.0).astype(jnp.bfloat16)

    fill3(sx1_ref, x_ref, 30, _W1P)
    s2_ref[:, :, 480:_W2P] = jnp.zeros((15, b, _W2P - 480), jnp.bfloat16)
    s2_ref[:, :, 0:480] = conv_pool(sx1_ref, w1_ref, b1_ref[...], 30, 480)
    fill3(sx2_ref, s2_ref, 12, _W2P)
    s3_ref[...] = conv_pool(sx2_ref, w2_ref, b2_ref[...], 12, 384)
    fill3(sx3_ref, s3_ref, 4, _W3P)
    p3 = conv_pool(sx3_ref, w3_ref, b3_ref[...], 4, 128)

    flat = jnp.concatenate([p3[0], p3[1]], axis=1)
    h = jnp.dot(flat, fw1_ref[...], preferred_element_type=jnp.float32)
    h = jnp.maximum(h + fb1_ref[...], 0.0).astype(jnp.bfloat16)
    logits = jnp.dot(h, fw2_ref[...].astype(jnp.bfloat16),
                     preferred_element_type=jnp.float32)
    o_ref[...] = logits + fb2_ref[...]


def kernel(w1, b1, w2, b2, w3, b3, fw1, fb1, fw2, fb2, x):
    n = x.shape[0]
    bsz = next(b for b in (128, 64, 32, 16, 8, 4, 2, 1) if n % b == 0)

    xp = jnp.transpose(x.astype(jnp.bfloat16), (2, 0, 3, 1))
    xp = jnp.pad(xp.reshape(_H1, n, _H1 * _C1), ((0, 0), (0, 0), (0, 32)))

    w1b = _pad_k(_banded(w1, _H1, _C1, _OC1), _W1P)
    w2b = _pad_k(_banded(w2, _H2, _C2, _OC2), _W2P)
    w3b = _pad_k(_banded(w3, _H3, _C3, _OC3), _W3P)
    b1t = jnp.tile(b1, (1, 15))
    b2t = jnp.tile(b2, (1, 6))
    b3t = jnp.tile(b3, (1, 2))
    fw1r = fw1.reshape(4 * _OC3, _FC1).astype(jnp.bfloat16)

    full2 = lambda i: (0, 0)
    out = pl.pallas_call(
        _cnn_kernel,
        out_shape=jax.ShapeDtypeStruct((n, _FC2), jnp.float32),
        grid_spec=pltpu.PrefetchScalarGridSpec(
            num_scalar_prefetch=0,
            grid=(n // bsz,),
            in_specs=[
                pl.BlockSpec((_H1, bsz, _W1P), lambda i: (0, i, 0)),
                pl.BlockSpec((3 * _W1P, 30 * _OC1), full2),
                pl.BlockSpec((1, 15 * _OC1), full2),
                pl.BlockSpec((3 * _W2P, 13 * _OC2), full2),
                pl.BlockSpec((1, 6 * _OC2), full2),
                pl.BlockSpec((3 * _W3P, 4 * _OC3), full2),
                pl.BlockSpec((1, 2 * _OC3), full2),
                pl.BlockSpec((4 * _OC3, _FC1), full2),
                pl.BlockSpec((1, _FC1), full2),
                pl.BlockSpec((_FC1, _FC2), full2),
                pl.BlockSpec((1, _FC2), full2),
            ],
            out_specs=pl.BlockSpec((bsz, _FC2), lambda i: (i, 0)),
            scratch_shapes=[
                pltpu.VMEM((30, bsz, 3 * _W1P), jnp.bfloat16),
                pltpu.VMEM((15, bsz, _W2P), jnp.bfloat16),
                pltpu.VMEM((12, bsz, 3 * _W2P), jnp.bfloat16),
                pltpu.VMEM((6, bsz, _W3P), jnp.bfloat16),
                pltpu.VMEM((4, bsz, 3 * _W3P), jnp.bfloat16),
            ],
        ),
        compiler_params=pltpu.CompilerParams(
            dimension_semantics=("parallel",)),
    )(xp, w1b, b1t, w2b, b2t, w3b, b3t, fw1r, fb1, fw2, fb2)
    return out


# speedup vs baseline: 1.3926x; 1.0147x over previous
import numpy as np

import jax
import jax.numpy as jnp
from jax.experimental import pallas as pl
from jax.experimental.pallas import tpu as pltpu

_H1, _C1, _OC1 = 32, 3, 32
_H2, _C2, _OC2 = 15, 32, 64
_H3, _C3, _OC3 = 6, 64, 64
_FC1, _FC2 = 128, 10
_W1P, _W2P, _W3P = 128, 512, 384


def _banded(w_taps, h, c, oc):
    ow = h - 2
    owp = ow // 2
    perm = [2 * k for k in range(owp)] + [2 * k + 1 for k in range(owp)]
    if ow % 2:
        perm.append(ow - 1)
    e = np.zeros((9, 3, h, ow), np.float32)
    for d in range(3):
        for dj in range(3):
            for jp, j in enumerate(perm):
                e[d * 3 + dj, d, j + dj, jp] = 1.0
    wb = jnp.einsum('tco,tdhj->dhcjo', w_taps, jnp.asarray(e))
    return wb.reshape(3, h * c, ow * oc)


def _pad_k(wb3, kp):
    k = wb3.shape[1]
    return jnp.pad(wb3, ((0, 0), (0, kp - k), (0, 0))).reshape(
        3 * kp, wb3.shape[2]).astype(jnp.bfloat16)


def _cnn_kernel(x_ref, w1_ref, b1_ref, w2_ref, b2_ref, w3_ref, b3_ref,
                fw1_ref, fb1_ref, fw2_ref, fb2_ref, o_ref,
                sx1_ref, s2_ref, sx2_ref, s3_ref, sx3_ref):
    b = x_ref.shape[1]

    def fill3(dst_ref, src_ref, rows, w):
        for di in range(3):
            dst_ref[:, :, di * w:(di + 1) * w] = src_ref[di:di + rows, :, :]

    def conv_pool(cat_ref, wb_ref, bias, rows, half):
        sl = cat_ref[...].reshape(rows * b, cat_ref.shape[2])
        acc = jnp.dot(sl, wb_ref[...], preferred_element_type=jnp.float32)
        n = acc.shape[-1]
        acc = acc.reshape(rows // 2, 2 * b, n)
        rm = jnp.maximum(acc[:, :b, :], acc[:, b:, :])
        cm = jnp.maximum(rm[..., :half], rm[..., half:2 * half])
        return jnp.maximum(cm + bias, 0.0).astype(jnp.bfloat16)

    fill3(sx1_ref, x_ref, 30, _W1P)
    s2_ref[:, :, 480:_W2P] = jnp.zeros((15, b, _W2P - 480), jnp.bfloat16)
    s2_ref[:, :, 0:480] = conv_pool(sx1_ref, w1_ref, b1_ref[...], 30, 480)
    fill3(sx2_ref, s2_ref, 12, _W2P)
    s3_ref[...] = conv_pool(sx2_ref, w2_ref, b2_ref[...], 12, 384)
    fill3(sx3_ref, s3_ref, 4, _W3P)
    p3 = conv_pool(sx3_ref, w3_ref, b3_ref[...], 4, 128)

    flat = jnp.concatenate([p3[0], p3[1]], axis=1)
    h = jnp.dot(flat, fw1_ref[...], preferred_element_type=jnp.float32)
    h = jnp.maximum(h + fb1_ref[...], 0.0).astype(jnp.bfloat16)
    logits = jnp.dot(h, fw2_ref[...].astype(jnp.bfloat16),
                     preferred_element_type=jnp.float32)
    o_ref[...] = logits + fb2_ref[...]


def kernel(w1, b1, w2, b2, w3, b3, fw1, fb1, fw2, fb2, x):
    n = x.shape[0]
    bsz = next(b for b in (256, 128, 64, 32, 16, 8, 4, 2, 1) if n % b == 0)

    xp = jnp.transpose(x.astype(jnp.bfloat16), (2, 0, 3, 1))
    xp = jnp.pad(xp.reshape(_H1, n, _H1 * _C1), ((0, 0), (0, 0), (0, 32)))

    w1b = _pad_k(_banded(w1, _H1, _C1, _OC1), _W1P)
    w2b = _pad_k(_banded(w2, _H2, _C2, _OC2), _W2P)
    w3b = _pad_k(_banded(w3, _H3, _C3, _OC3), _W3P)
    b1t = jnp.tile(b1, (1, 15))
    b2t = jnp.tile(b2, (1, 6))
    b3t = jnp.tile(b3, (1, 2))
    fw1r = fw1.reshape(4 * _OC3, _FC1).astype(jnp.bfloat16)

    full2 = lambda i: (0, 0)
    out = pl.pallas_call(
        _cnn_kernel,
        out_shape=jax.ShapeDtypeStruct((n, _FC2), jnp.float32),
        grid_spec=pltpu.PrefetchScalarGridSpec(
            num_scalar_prefetch=0,
            grid=(n // bsz,),
            in_specs=[
                pl.BlockSpec((_H1, bsz, _W1P), lambda i: (0, i, 0)),
                pl.BlockSpec((3 * _W1P, 30 * _OC1), full2),
                pl.BlockSpec((1, 15 * _OC1), full2),
                pl.BlockSpec((3 * _W2P, 13 * _OC2), full2),
                pl.BlockSpec((1, 6 * _OC2), full2),
                pl.BlockSpec((3 * _W3P, 4 * _OC3), full2),
                pl.BlockSpec((1, 2 * _OC3), full2),
                pl.BlockSpec((4 * _OC3, _FC1), full2),
                pl.BlockSpec((1, _FC1), full2),
                pl.BlockSpec((_FC1, _FC2), full2),
                pl.BlockSpec((1, _FC2), full2),
            ],
            out_specs=pl.BlockSpec((bsz, _FC2), lambda i: (i, 0)),
            scratch_shapes=[
                pltpu.VMEM((30, bsz, 3 * _W1P), jnp.bfloat16),
                pltpu.VMEM((15, bsz, _W2P), jnp.bfloat16),
                pltpu.VMEM((12, bsz, 3 * _W2P), jnp.bfloat16),
                pltpu.VMEM((6, bsz, _W3P), jnp.bfloat16),
                pltpu.VMEM((4, bsz, 3 * _W3P), jnp.bfloat16),
            ],
        ),
        compiler_params=pltpu.CompilerParams(
            dimension_semantics=("parallel",)),
    )(xp, w1b, b1t, w2b, b2t, w3b, b3t, fw1r, fb1, fw2, fb2)
    return out


# speedup vs baseline: 1.4003x; 1.0055x over previous
import numpy as np

import jax
import jax.numpy as jnp
from jax.experimental import pallas as pl
from jax.experimental.pallas import tpu as pltpu

_H1, _C1, _OC1 = 32, 3, 32
_H2, _C2, _OC2 = 15, 32, 64
_H3, _C3, _OC3 = 6, 64, 64
_FC1, _FC2 = 128, 10
_W1P, _W2P, _W3P = 128, 512, 384


def _banded(w_taps, h, c, oc):
    ow = h - 2
    owp = ow // 2
    perm = [2 * k for k in range(owp)] + [2 * k + 1 for k in range(owp)]
    e = np.zeros((9, 3, h, len(perm)), np.float32)
    for d in range(3):
        for dj in range(3):
            for jp, j in enumerate(perm):
                e[d * 3 + dj, d, j + dj, jp] = 1.0
    wb = jnp.einsum('tco,tdhj->dhcjo', w_taps, jnp.asarray(e))
    return wb.reshape(3, h * c, len(perm) * oc)


def _pad_k(wb3, kp):
    k = wb3.shape[1]
    return jnp.pad(wb3, ((0, 0), (0, kp - k), (0, 0))).reshape(
        3 * kp, wb3.shape[2]).astype(jnp.bfloat16)


def _cnn_kernel(x_ref, w1_ref, b1_ref, w2_ref, b2_ref, w3_ref, b3_ref,
                fw1_ref, fb1_ref, fw2_ref, fb2_ref, o_ref,
                sx1_ref, s2_ref, sx2_ref, s3_ref, sx3_ref):
    b = x_ref.shape[1]

    def fill3(dst_ref, src_ref, rows, w):
        for di in range(3):
            dst_ref[:, :, di * w:(di + 1) * w] = src_ref[di:di + rows, :, :]

    def conv_pool(cat_ref, wb_ref, bias, rows, half):
        sl = cat_ref[...].reshape(rows * b, cat_ref.shape[2])
        acc = jnp.dot(sl, wb_ref[...], preferred_element_type=jnp.float32)
        n = acc.shape[-1]
        acc = acc.reshape(rows // 2, 2 * b, n)
        rm = jnp.maximum(acc[:, :b, :], acc[:, b:, :])
        cm = jnp.maximum(rm[..., :half], rm[..., half:2 * half])
        return jnp.maximum(cm + bias, 0.0).astype(jnp.bfloat16)

    fill3(sx1_ref, x_ref, 30, _W1P)
    s2_ref[:, :, 480:_W2P] = jnp.zeros((15, b, _W2P - 480), jnp.bfloat16)
    s2_ref[:, :, 0:480] = conv_pool(sx1_ref, w1_ref, b1_ref[...], 30, 480)
    fill3(sx2_ref, s2_ref, 12, _W2P)
    s3_ref[...] = conv_pool(sx2_ref, w2_ref, b2_ref[...], 12, 384)
    fill3(sx3_ref, s3_ref, 4, _W3P)
    p3 = conv_pool(sx3_ref, w3_ref, b3_ref[...], 4, 128)

    flat = jnp.concatenate([p3[0], p3[1]], axis=1)
    h = jnp.dot(flat, fw1_ref[...], preferred_element_type=jnp.float32)
    h = jnp.maximum(h + fb1_ref[...], 0.0).astype(jnp.bfloat16)
    logits = jnp.dot(h, fw2_ref[...],
                     preferred_element_type=jnp.float32)
    o_ref[...] = logits + fb2_ref[...]


def kernel(w1, b1, w2, b2, w3, b3, fw1, fb1, fw2, fb2, x):
    n = x.shape[0]
    bsz = next(b for b in (256, 128, 64, 32, 16, 8, 4, 2, 1) if n % b == 0)

    xp = jnp.transpose(x.astype(jnp.bfloat16), (2, 0, 3, 1))
    xp = jnp.pad(xp.reshape(_H1, n, _H1 * _C1), ((0, 0), (0, 0), (0, 32)))

    w1b = _pad_k(_banded(w1, _H1, _C1, _OC1), _W1P)
    w2b = _pad_k(_banded(w2, _H2, _C2, _OC2), _W2P)
    w3b = _pad_k(_banded(w3, _H3, _C3, _OC3), _W3P)
    b1t = jnp.tile(b1, (1, 15))
    b2t = jnp.tile(b2, (1, 6))
    b3t = jnp.tile(b3, (1, 2))
    fw1r = fw1.reshape(4 * _OC3, _FC1).astype(jnp.bfloat16)
    fw2c = fw2.astype(jnp.bfloat16)

    full2 = lambda i: (0, 0)
    out = pl.pallas_call(
        _cnn_kernel,
        out_shape=jax.ShapeDtypeStruct((n, _FC2), jnp.float32),
        grid_spec=pltpu.PrefetchScalarGridSpec(
            num_scalar_prefetch=0,
            grid=(n // bsz,),
            in_specs=[
                pl.BlockSpec((_H1, bsz, _W1P), lambda i: (0, i, 0)),
                pl.BlockSpec((3 * _W1P, 30 * _OC1), full2),
                pl.BlockSpec((1, 15 * _OC1), full2),
                pl.BlockSpec((3 * _W2P, 12 * _OC2), full2),
                pl.BlockSpec((1, 6 * _OC2), full2),
                pl.BlockSpec((3 * _W3P, 4 * _OC3), full2),
                pl.BlockSpec((1, 2 * _OC3), full2),
                pl.BlockSpec((4 * _OC3, _FC1), full2),
                pl.BlockSpec((1, _FC1), full2),
                pl.BlockSpec((_FC1, _FC2), full2),
                pl.BlockSpec((1, _FC2), full2),
            ],
            out_specs=pl.BlockSpec((bsz, _FC2), lambda i: (i, 0)),
            scratch_shapes=[
                pltpu.VMEM((30, bsz, 3 * _W1P), jnp.bfloat16),
                pltpu.VMEM((15, bsz, _W2P), jnp.bfloat16),
                pltpu.VMEM((12, bsz, 3 * _W2P), jnp.bfloat16),
                pltpu.VMEM((6, bsz, _W3P), jnp.bfloat16),
                pltpu.VMEM((4, bsz, 3 * _W3P), jnp.bfloat16),
            ],
        ),
        compiler_params=pltpu.CompilerParams(
            dimension_semantics=("parallel",)),
    )(xp, w1b, b1t, w2b, b2t, w3b, b3t, fw1r, fb1, fw2c, fb2)
    return out


# speedup vs baseline: 1.4394x; 1.0279x over previous
import numpy as np

import jax
import jax.numpy as jnp
from jax.experimental import pallas as pl
from jax.experimental.pallas import tpu as pltpu

_H1, _C1, _OC1 = 32, 3, 32
_H2, _C2, _OC2 = 15, 32, 64
_H3, _C3, _OC3 = 6, 64, 64
_FC1, _FC2 = 128, 10
_W1P, _W2P, _W3P = 96, 512, 384


def _banded(w_taps, h, c, oc):
    ow = h - 2
    owp = ow // 2
    perm = [2 * k for k in range(owp)] + [2 * k + 1 for k in range(owp)]
    e = np.zeros((9, 3, h, len(perm)), np.float32)
    for d in range(3):
        for dj in range(3):
            for jp, j in enumerate(perm):
                e[d * 3 + dj, d, j + dj, jp] = 1.0
    wb = jnp.einsum('tco,tdhj->dhcjo', w_taps, jnp.asarray(e))
    return wb.reshape(3, h * c, len(perm) * oc)


def _pad_k(wb3, kp):
    k = wb3.shape[1]
    return jnp.pad(wb3, ((0, 0), (0, kp - k), (0, 0))).reshape(
        3 * kp, wb3.shape[2]).astype(jnp.bfloat16)


def _cnn_kernel(x_ref, w1_ref, b1_ref, w2_ref, b2_ref, w3_ref, b3_ref,
                fw1_ref, fb1_ref, fw2_ref, fb2_ref, o_ref,
                sx1_ref, s2_ref, sx2_ref, s3_ref, sx3_ref):
    b = x_ref.shape[1]

    def fill3(dst_ref, src_ref, rows, w):
        for di in range(3):
            dst_ref[:, :, di * w:(di + 1) * w] = src_ref[di:di + rows, :, :]

    def conv_pool(cat_ref, wb_ref, bias, rows, half):
        sl = cat_ref[...].reshape(rows * b, cat_ref.shape[2])
        acc = jnp.dot(sl, wb_ref[...], preferred_element_type=jnp.float32)
        n = acc.shape[-1]
        acc = acc.reshape(rows // 2, 2 * b, n)
        rm = jnp.maximum(acc[:, :b, :], acc[:, b:, :])
        cm = jnp.maximum(rm[..., :half], rm[..., half:2 * half])
        return jnp.maximum(cm + bias, 0.0).astype(jnp.bfloat16)

    fill3(sx1_ref, x_ref, 30, _W1P)
    s2_ref[:, :, 480:_W2P] = jnp.zeros((15, b, _W2P - 480), jnp.bfloat16)
    s2_ref[:, :, 0:480] = conv_pool(sx1_ref, w1_ref, b1_ref[...], 30, 480)
    fill3(sx2_ref, s2_ref, 12, _W2P)
    s3_ref[...] = conv_pool(sx2_ref, w2_ref, b2_ref[...], 12, 384)
    fill3(sx3_ref, s3_ref, 4, _W3P)
    p3 = conv_pool(sx3_ref, w3_ref, b3_ref[...], 4, 128)

    flat = jnp.concatenate([p3[0], p3[1]], axis=1)
    h = jnp.dot(flat, fw1_ref[...], preferred_element_type=jnp.float32)
    h = jnp.maximum(h + fb1_ref[...], 0.0).astype(jnp.bfloat16)
    logits = jnp.dot(h, fw2_ref[...],
                     preferred_element_type=jnp.float32)
    o_ref[...] = logits + fb2_ref[...]


def kernel(w1, b1, w2, b2, w3, b3, fw1, fb1, fw2, fb2, x):
    n = x.shape[0]
    bsz = next(b for b in (256, 128, 64, 32, 16, 8, 4, 2, 1) if n % b == 0)

    xp = jnp.transpose(x.astype(jnp.bfloat16), (2, 0, 3, 1))
    xp = xp.reshape(_H1, n, _H1 * _C1)

    w1b = _pad_k(_banded(w1, _H1, _C1, _OC1), _W1P)
    w2b = _pad_k(_banded(w2, _H2, _C2, _OC2), _W2P)
    w3b = _pad_k(_banded(w3, _H3, _C3, _OC3), _W3P)
    b1t = jnp.tile(b1, (1, 15))
    b2t = jnp.tile(b2, (1, 6))
    b3t = jnp.tile(b3, (1, 2))
    fw1r = fw1.reshape(4 * _OC3, _FC1).astype(jnp.bfloat16)
    fw2c = fw2.astype(jnp.bfloat16)

    full2 = lambda i: (0, 0)
    out = pl.pallas_call(
        _cnn_kernel,
        out_shape=jax.ShapeDtypeStruct((n, _FC2), jnp.float32),
        grid_spec=pltpu.PrefetchScalarGridSpec(
            num_scalar_prefetch=0,
            grid=(n // bsz,),
            in_specs=[
                pl.BlockSpec((_H1, bsz, _W1P), lambda i: (0, i, 0)),
                pl.BlockSpec((3 * _W1P, 30 * _OC1), full2),
                pl.BlockSpec((1, 15 * _OC1), full2),
                pl.BlockSpec((3 * _W2P, 12 * _OC2), full2),
                pl.BlockSpec((1, 6 * _OC2), full2),
                pl.BlockSpec((3 * _W3P, 4 * _OC3), full2),
                pl.BlockSpec((1, 2 * _OC3), full2),
                pl.BlockSpec((4 * _OC3, _FC1), full2),
                pl.BlockSpec((1, _FC1), full2),
                pl.BlockSpec((_FC1, _FC2), full2),
                pl.BlockSpec((1, _FC2), full2),
            ],
            out_specs=pl.BlockSpec((bsz, _FC2), lambda i: (i, 0)),
            scratch_shapes=[
                pltpu.VMEM((30, bsz, 3 * _W1P), jnp.bfloat16),
                pltpu.VMEM((15, bsz, _W2P), jnp.bfloat16),
                pltpu.VMEM((12, bsz, 3 * _W2P), jnp.bfloat16),
                pltpu.VMEM((6, bsz, _W3P), jnp.bfloat16),
                pltpu.VMEM((4, bsz, 3 * _W3P), jnp.bfloat16),
            ],
        ),
        compiler_params=pltpu.CompilerParams(
            dimension_semantics=("parallel",)),
    )(xp, w1b, b1t, w2b, b2t, w3b, b3t, fw1r, fb1, fw2c, fb2)
    return out
